# Initial kernel scaffold; baseline (speedup 1.0000x reference)
import jax
import jax.numpy as jnp
from jax.experimental import pallas as pl


def kernel(x, L_hgnn_idx, L_hgnn_val, L_sym_idx, L_sym_val, L_rw_idx, L_rw_val, W_weight, W_bias):
    raise NotImplementedError("write your pallas kernel here")


# speedup vs baseline: 3.8643x; 3.8643x over previous
import functools

import jax
import jax.numpy as jnp
from jax import lax
from jax.experimental import pallas as pl
from jax.experimental.pallas import tpu as pltpu
from jax.experimental.pallas import tpu_sc as plsc

N = 10000
D = 128
OUT = 128
E = 320000

NC = 2
NS = 16
NW = NC * NS
C = 128

E3 = 3 * E
CHUNKS_PER_W = -(-E3 // (NW * C))
EPW = CHUNKS_PER_W * C
ET = EPW * NW
NP = 10240
ROWS_PER_TILE = NP // NS


def _mm_body(x_ref, wt_ref, o_ref):
    o_ref[0] = jnp.dot(x_ref[...], wt_ref[0], preferred_element_type=jnp.float32)


def _make_table(x, wt):
    bn = 1000
    return pl.pallas_call(
        _mm_body,
        grid=(2, N // bn),
        in_specs=[
            pl.BlockSpec((bn, D), lambda w, i: (i, 0)),
            pl.BlockSpec((1, D, OUT), lambda w, i: (w, 0, 0)),
        ],
        out_specs=pl.BlockSpec((1, bn, OUT), lambda w, i: (w, i, 0)),
        out_shape=jax.ShapeDtypeStruct((2, N, OUT), jnp.float32),
    )(x, wt)


def _fin_body(p_ref, b_ref, o_ref):
    o_ref[...] = jnp.maximum(p_ref[0] + p_ref[1] + b_ref[...], 0.0)


def _finish(part, bias2d):
    bn = 1000
    return pl.pallas_call(
        _fin_body,
        grid=(N // bn,),
        in_specs=[
            pl.BlockSpec((2, bn, OUT), lambda i: (0, i, 0)),
            pl.BlockSpec((1, OUT), lambda i: (0, 0)),
        ],
        out_specs=pl.BlockSpec((bn, OUT), lambda i: (i, 0)),
        out_shape=jax.ShapeDtypeStruct((N, OUT), jnp.float32),
    )(part, bias2d)


def _sc_body(table_hbm, dst_hbm, src_hbm, val_hbm, part_hbm,
             dstv, srcv, valv, rows, acc, sem):
    c = lax.axis_index("c")
    s = lax.axis_index("s")
    wid = c * NS + s

    def zrow(i, carry):
        for j in range(8):
            rows[i, pl.ds(16 * j, 16)] = jnp.zeros((16,), jnp.float32)
        return carry
    lax.fori_loop(0, C, zrow, 0)
    for k in range(ROWS_PER_TILE // C):
        pltpu.sync_copy(rows,
                        acc.at[pl.ds(ROWS_PER_TILE * s + C * k, C)])
    plsc.subcore_barrier()

    base0 = wid * EPW

    def chunk(it, carry):
        base = base0 + it * C
        pltpu.sync_copy(dst_hbm.at[pl.ds(base, C)], dstv)
        pltpu.sync_copy(src_hbm.at[pl.ds(base, C)], srcv)
        pltpu.sync_copy(val_hbm.at[pl.ds(base, C)], valv)
        pltpu.async_copy(table_hbm.at[srcv], rows, sem).wait()

        def scale(g, cc):
            vvec = valv[pl.ds(16 * g, 16)]
            for l in range(16):
                vv = jnp.take_along_axis(
                    vvec, jnp.full((16,), l, jnp.int32), axis=0)
                i = 16 * g + l
                for j in range(8):
                    sl = pl.ds(16 * j, 16)
                    rows[i, sl] = rows[i, sl] * vv
            return cc
        lax.fori_loop(0, C // 16, scale, 0)
        pltpu.sync_copy(rows, acc.at[dstv], add=True)
        return carry
    lax.fori_loop(0, CHUNKS_PER_W, chunk, 0)
    plsc.subcore_barrier()

    pltpu.sync_copy(acc.at[pl.ds(ROWS_PER_TILE * s, ROWS_PER_TILE)],
                    part_hbm.at[c, pl.ds(ROWS_PER_TILE * s, ROWS_PER_TILE)])


_sc_spmm = pl.kernel(
    _sc_body,
    out_type=jax.ShapeDtypeStruct((2, NP, OUT), jnp.float32),
    mesh=plsc.VectorSubcoreMesh(core_axis_name="c", subcore_axis_name="s"),
    scratch_types=[
        pltpu.VMEM((C,), jnp.int32),
        pltpu.VMEM((C,), jnp.int32),
        pltpu.VMEM((C,), jnp.float32),
        pltpu.VMEM((C, OUT), jnp.float32),
        pltpu.VMEM_SHARED((NP, OUT), jnp.float32),
        pltpu.SemaphoreType.DMA,
    ],
)


def kernel(x, L_hgnn_idx, L_hgnn_val, L_sym_idx, L_sym_val, L_rw_idx, L_rw_val,
           W_weight, W_bias):
    wt = jnp.stack([W_weight[:, :D].T, 0.5 * W_weight[:, D:].T])
    table = _make_table(x, wt).reshape(2 * N, OUT)

    pad = ET - E3
    dst = jnp.concatenate([
        L_hgnn_idx[0], L_sym_idx[0], L_rw_idx[0],
        jnp.zeros((pad,), jnp.int32)])
    src = jnp.concatenate([
        L_hgnn_idx[1], L_sym_idx[1] + N, L_rw_idx[1] + N,
        jnp.zeros((pad,), jnp.int32)])
    val = jnp.concatenate([
        L_hgnn_val, L_sym_val, L_rw_val,
        jnp.zeros((pad,), jnp.float32)])

    part = _sc_spmm(table, dst, src, val)
    return _finish(part, W_bias.reshape(1, OUT))


# speedup vs baseline: 5.5121x; 1.4264x over previous
import functools

import jax
import jax.numpy as jnp
from jax import lax
from jax.experimental import pallas as pl
from jax.experimental.pallas import tpu as pltpu
from jax.experimental.pallas import tpu_sc as plsc

N = 10000
D = 128
OUT = 128
E = 320000

NC = 2
NS = 16
NW = NC * NS
C = 128

E3 = 3 * E
CHUNKS_PER_W = 236
EPW = CHUNKS_PER_W * C
ET = EPW * NW
NP = 10240
ROWS_PER_TILE = NP // NS


def _mm_body(x_ref, wt_ref, o_ref):
    o_ref[0] = jnp.dot(x_ref[...], wt_ref[0], preferred_element_type=jnp.float32)


def _make_table(x, wt):
    bn = 1000
    return pl.pallas_call(
        _mm_body,
        grid=(2, N // bn),
        in_specs=[
            pl.BlockSpec((bn, D), lambda w, i: (i, 0)),
            pl.BlockSpec((1, D, OUT), lambda w, i: (w, 0, 0)),
        ],
        out_specs=pl.BlockSpec((1, bn, OUT), lambda w, i: (w, i, 0)),
        out_shape=jax.ShapeDtypeStruct((2, N, OUT), jnp.float32),
    )(x, wt)


def _fin_body(p_ref, b_ref, o_ref):
    o_ref[...] = jnp.maximum(p_ref[0] + p_ref[1] + b_ref[...], 0.0)


def _finish(part, bias2d):
    bn = 1000
    return pl.pallas_call(
        _fin_body,
        grid=(N // bn,),
        in_specs=[
            pl.BlockSpec((2, bn, OUT), lambda i: (0, i, 0)),
            pl.BlockSpec((1, OUT), lambda i: (0, 0)),
        ],
        out_specs=pl.BlockSpec((bn, OUT), lambda i: (i, 0)),
        out_shape=jax.ShapeDtypeStruct((N, OUT), jnp.float32),
    )(part, bias2d)


def _sc_body(table_hbm, dst_hbm, src_hbm, val_hbm, part_hbm,
             dstv0, srcv0, valv0, dstv1, srcv1, valv1,
             rows0, rows1, acc, semg0, semg1, semi0, semi1):
    c = lax.axis_index("c")
    s = lax.axis_index("s")
    wid = c * NS + s

    def zrow(i, carry):
        for j in range(8):
            rows0[i, pl.ds(16 * j, 16)] = jnp.zeros((16,), jnp.float32)
        return carry
    lax.fori_loop(0, C, zrow, 0)
    for k in range(ROWS_PER_TILE // C):
        pltpu.sync_copy(rows0,
                        acc.at[pl.ds(ROWS_PER_TILE * s + C * k, C)])
    plsc.subcore_barrier()

    base0 = wid * EPW

    def load_idx(k, dstv, srcv, valv, semi):
        base = base0 + k * C
        pltpu.async_copy(dst_hbm.at[pl.ds(base, C)], dstv, semi)
        pltpu.async_copy(src_hbm.at[pl.ds(base, C)], srcv, semi)
        pltpu.async_copy(val_hbm.at[pl.ds(base, C)], valv, semi)

    def wait_idx(dstv, srcv, valv, semi):
        pltpu.make_async_copy(dst_hbm.at[pl.ds(0, C)], dstv, semi).wait()
        pltpu.make_async_copy(src_hbm.at[pl.ds(0, C)], srcv, semi).wait()
        pltpu.make_async_copy(val_hbm.at[pl.ds(0, C)], valv, semi).wait()

    def scale_scatter(rows, dstv, valv):
        def scale(g, cc):
            vvec = valv[pl.ds(16 * g, 16)]
            for l in range(16):
                vv = jnp.take_along_axis(
                    vvec, jnp.full((16,), l, jnp.int32), axis=0)
                i = 16 * g + l
                for j in range(8):
                    sl = pl.ds(16 * j, 16)
                    rows[i, sl] = rows[i, sl] * vv
            return cc
        lax.fori_loop(0, C // 16, scale, 0)
        pltpu.sync_copy(rows, acc.at[dstv], add=True)

    load_idx(0, dstv0, srcv0, valv0, semi0)
    wait_idx(dstv0, srcv0, valv0, semi0)
    pltpu.async_copy(table_hbm.at[srcv0], rows0, semg0)
    load_idx(1, dstv1, srcv1, valv1, semi1)
    wait_idx(dstv1, srcv1, valv1, semi1)
    pltpu.async_copy(table_hbm.at[srcv1], rows1, semg1)

    def pair(i, carry):
        k = 2 * i
        pltpu.make_async_copy(table_hbm.at[srcv0], rows0, semg0).wait()
        scale_scatter(rows0, dstv0, valv0)

        @pl.when(k + 2 < CHUNKS_PER_W)
        def _():
            load_idx(k + 2, dstv0, srcv0, valv0, semi0)
            wait_idx(dstv0, srcv0, valv0, semi0)
            pltpu.async_copy(table_hbm.at[srcv0], rows0, semg0)

        pltpu.make_async_copy(table_hbm.at[srcv1], rows1, semg1).wait()
        scale_scatter(rows1, dstv1, valv1)

        @pl.when(k + 3 < CHUNKS_PER_W)
        def _():
            load_idx(k + 3, dstv1, srcv1, valv1, semi1)
            wait_idx(dstv1, srcv1, valv1, semi1)
            pltpu.async_copy(table_hbm.at[srcv1], rows1, semg1)
        return carry
    lax.fori_loop(0, CHUNKS_PER_W // 2, pair, 0)
    plsc.subcore_barrier()

    pltpu.sync_copy(acc.at[pl.ds(ROWS_PER_TILE * s, ROWS_PER_TILE)],
                    part_hbm.at[c, pl.ds(ROWS_PER_TILE * s, ROWS_PER_TILE)])


_sc_spmm = pl.kernel(
    _sc_body,
    out_type=jax.ShapeDtypeStruct((2, NP, OUT), jnp.float32),
    mesh=plsc.VectorSubcoreMesh(core_axis_name="c", subcore_axis_name="s"),
    scratch_types=[
        pltpu.VMEM((C,), jnp.int32),
        pltpu.VMEM((C,), jnp.int32),
        pltpu.VMEM((C,), jnp.float32),
        pltpu.VMEM((C,), jnp.int32),
        pltpu.VMEM((C,), jnp.int32),
        pltpu.VMEM((C,), jnp.float32),
        pltpu.VMEM((C, OUT), jnp.float32),
        pltpu.VMEM((C, OUT), jnp.float32),
        pltpu.VMEM_SHARED((NP, OUT), jnp.float32),
        pltpu.SemaphoreType.DMA,
        pltpu.SemaphoreType.DMA,
        pltpu.SemaphoreType.DMA,
        pltpu.SemaphoreType.DMA,
    ],
)


def kernel(x, L_hgnn_idx, L_hgnn_val, L_sym_idx, L_sym_val, L_rw_idx, L_rw_val,
           W_weight, W_bias):
    wt = jnp.stack([W_weight[:, :D].T, 0.5 * W_weight[:, D:].T])
    table = _make_table(x, wt).reshape(2 * N, OUT)

    pad = ET - E3
    dst = jnp.concatenate([
        L_hgnn_idx[0], L_sym_idx[0], L_rw_idx[0],
        jnp.zeros((pad,), jnp.int32)])
    src = jnp.concatenate([
        L_hgnn_idx[1], L_sym_idx[1] + N, L_rw_idx[1] + N,
        jnp.zeros((pad,), jnp.int32)])
    val = jnp.concatenate([
        L_hgnn_val, L_sym_val, L_rw_val,
        jnp.zeros((pad,), jnp.float32)])

    part = _sc_spmm(table, dst, src, val)
    return _finish(part, W_bias.reshape(1, OUT))
